# baseline (device time: 17436 ns/iter reference)
import jax
import jax.numpy as jnp
from jax import lax
from jax.experimental import pallas as pl
from jax.experimental.pallas import tpu as pltpu


def kernel(x):
    m, n = x.shape

    def body(x_ref, out_ref, send_sem, recv_sem, copy_sem):
        my_x = lax.axis_index("x")
        my_y = lax.axis_index("y")
        my_z = lax.axis_index("z")
        peer = (1 - my_x, my_y, my_z)

        barrier_sem = pltpu.get_barrier_semaphore()
        pl.semaphore_signal(
            barrier_sem,
            inc=1,
            device_id=peer,
            device_id_type=pl.DeviceIdType.MESH,
        )
        pl.semaphore_wait(barrier_sem, 1)

        rdma = pltpu.make_async_remote_copy(
            src_ref=x_ref,
            dst_ref=out_ref.at[pl.ds(my_x * m, m), :],
            send_sem=send_sem,
            recv_sem=recv_sem,
            device_id=peer,
            device_id_type=pl.DeviceIdType.MESH,
        )
        rdma.start()

        local = pltpu.make_async_copy(
            x_ref, out_ref.at[pl.ds(my_x * m, m), :], copy_sem
        )
        local.start()
        local.wait()

        rdma.wait()

    return pl.pallas_call(
        body,
        out_shape=jax.ShapeDtypeStruct((2 * m, n), x.dtype),
        in_specs=[pl.BlockSpec(memory_space=pl.ANY)],
        out_specs=pl.BlockSpec(memory_space=pl.ANY),
        scratch_shapes=[
            pltpu.SemaphoreType.DMA,
            pltpu.SemaphoreType.DMA,
            pltpu.SemaphoreType.DMA,
        ],
        compiler_params=pltpu.CompilerParams(collective_id=0),
    )(x)


# device time: 15553 ns/iter; 1.1211x vs baseline; 1.1211x over previous
import jax
import jax.numpy as jnp
from jax import lax
from jax.experimental import pallas as pl
from jax.experimental.pallas import tpu as pltpu

K = 4


def kernel(x):
    m, n = x.shape
    half = m // 2
    ck = half // K

    def body(x_ref, out_ref, x_send, x_recv, z_send, z_recv, copy_sem):
        my_x = lax.axis_index("x")
        my_y = lax.axis_index("y")
        my_z = lax.axis_index("z")
        xpeer = (1 - my_x, my_y, my_z)
        zpart = (my_x, my_y, my_z ^ 1)
        sel = my_z % 2
        mybase = my_x * m
        rembase = (1 - my_x) * m

        barrier_sem = pltpu.get_barrier_semaphore()
        for tgt in (xpeer, zpart):
            pl.semaphore_signal(
                barrier_sem, inc=1, device_id=tgt,
                device_id_type=pl.DeviceIdType.MESH,
            )
        pl.semaphore_wait(barrier_sem, 2)

        local = pltpu.make_async_copy(
            x_ref, out_ref.at[pl.ds(mybase, m), :], copy_sem
        )
        local.start()

        x_rdmas = []
        for c in range(K):
            off = c * ck
            rdma = pltpu.make_async_remote_copy(
                src_ref=x_ref.at[pl.ds(sel * half + off, ck), :],
                dst_ref=out_ref.at[pl.ds(mybase + sel * half + off, ck), :],
                send_sem=x_send.at[c],
                recv_sem=x_recv.at[c],
                device_id=xpeer,
                device_id_type=pl.DeviceIdType.MESH,
            )
            rdma.start()
            x_rdmas.append(rdma)

        z_rdmas = []
        for c in range(K):
            off = c * ck
            x_rdmas[c].wait_recv()
            rdma = pltpu.make_async_remote_copy(
                src_ref=out_ref.at[pl.ds(rembase + sel * half + off, ck), :],
                dst_ref=out_ref.at[pl.ds(rembase + sel * half + off, ck), :],
                send_sem=z_send.at[c],
                recv_sem=z_recv.at[c],
                device_id=zpart,
                device_id_type=pl.DeviceIdType.MESH,
            )
            rdma.start()
            z_rdmas.append(rdma)

        for c in range(K):
            z_rdmas[c].wait_recv()
        for c in range(K):
            x_rdmas[c].wait_send()
            z_rdmas[c].wait_send()
        local.wait()

    return pl.pallas_call(
        body,
        out_shape=jax.ShapeDtypeStruct((2 * m, n), x.dtype),
        in_specs=[pl.BlockSpec(memory_space=pltpu.VMEM)],
        out_specs=pl.BlockSpec(memory_space=pltpu.VMEM),
        scratch_shapes=[
            pltpu.SemaphoreType.DMA((K,)),
            pltpu.SemaphoreType.DMA((K,)),
            pltpu.SemaphoreType.DMA((K,)),
            pltpu.SemaphoreType.DMA((K,)),
            pltpu.SemaphoreType.DMA,
        ],
        compiler_params=pltpu.CompilerParams(collective_id=0),
    )(x)


# device time: 14674 ns/iter; 1.1882x vs baseline; 1.0599x over previous
import jax
import jax.numpy as jnp
from jax import lax
from jax.experimental import pallas as pl
from jax.experimental.pallas import tpu as pltpu

R = 48
FWD_CHUNKS = ((0, 56), (56, 56), (112, 48), (160, 48))
HEAD = 208


def kernel(x):
    m, n = x.shape
    half = m // 2
    assert HEAD + R == half

    def body(x_ref, out_ref, x_send, x_recv, z_send, z_recv, copy_sem):
        my_x = lax.axis_index("x")
        my_y = lax.axis_index("y")
        my_z = lax.axis_index("z")
        xpeer = (1 - my_x, my_y, my_z)
        zpart = (my_x, my_y, my_z ^ 1)
        sel = my_z % 2
        mybase = my_x * m
        rembase = (1 - my_x) * m
        sb = sel * half
        cb = (1 - sel) * half

        barrier_sem = pltpu.get_barrier_semaphore()
        for tgt in (xpeer, zpart):
            pl.semaphore_signal(
                barrier_sem, inc=1, device_id=tgt,
                device_id_type=pl.DeviceIdType.MESH,
            )
        pl.semaphore_wait(barrier_sem, 2)

        local = pltpu.make_async_copy(
            x_ref, out_ref.at[pl.ds(mybase, m), :], copy_sem
        )
        local.start()

        x_regions = [(sb + off, rows) for off, rows in FWD_CHUNKS]
        x_regions.append((sb + HEAD, R))
        x_regions.append((cb + HEAD, R))
        x_rdmas = []
        for c, (off, rows) in enumerate(x_regions):
            rdma = pltpu.make_async_remote_copy(
                src_ref=x_ref.at[pl.ds(off, rows), :],
                dst_ref=out_ref.at[pl.ds(mybase + off, rows), :],
                send_sem=x_send.at[c],
                recv_sem=x_recv.at[c],
                device_id=xpeer,
                device_id_type=pl.DeviceIdType.MESH,
            )
            rdma.start()
            x_rdmas.append(rdma)

        z_rdmas = []
        for c, (off, rows) in enumerate(FWD_CHUNKS):
            x_rdmas[c].wait_recv()
            rdma = pltpu.make_async_remote_copy(
                src_ref=out_ref.at[pl.ds(rembase + sb + off, rows), :],
                dst_ref=out_ref.at[pl.ds(rembase + sb + off, rows), :],
                send_sem=z_send.at[c],
                recv_sem=z_recv.at[c],
                device_id=zpart,
                device_id_type=pl.DeviceIdType.MESH,
            )
            rdma.start()
            z_rdmas.append(rdma)

        for c in range(len(FWD_CHUNKS), len(x_regions)):
            x_rdmas[c].wait_recv()
        for rdma in z_rdmas:
            rdma.wait_recv()
        for rdma in x_rdmas:
            rdma.wait_send()
        for rdma in z_rdmas:
            rdma.wait_send()
        local.wait()

    n_x = len(FWD_CHUNKS) + 2
    n_z = len(FWD_CHUNKS)
    return pl.pallas_call(
        body,
        out_shape=jax.ShapeDtypeStruct((2 * m, n), x.dtype),
        in_specs=[pl.BlockSpec(memory_space=pltpu.VMEM)],
        out_specs=pl.BlockSpec(memory_space=pltpu.VMEM),
        scratch_shapes=[
            pltpu.SemaphoreType.DMA((n_x,)),
            pltpu.SemaphoreType.DMA((n_x,)),
            pltpu.SemaphoreType.DMA((n_z,)),
            pltpu.SemaphoreType.DMA((n_z,)),
            pltpu.SemaphoreType.DMA,
        ],
        compiler_params=pltpu.CompilerParams(collective_id=0),
    )(x)
